# Initial kernel scaffold; baseline (speedup 1.0000x reference)
import jax
import jax.numpy as jnp
from jax import lax
from jax.experimental import pallas as pl


def _tsd_kernel(d1_ref, d2_ref, dist_ref, matched_ref, d12_ref, d21_ref):
    n, S, L = d1_ref.shape
    iota = (lax.broadcasted_iota(jnp.int32, (S, L), 0) * L
            + lax.broadcasted_iota(jnp.int32, (S, L), 1))
    INF = jnp.float32(jnp.inf)
    BIG = jnp.int32(1 << 30)
    root = iota == 0

    def step(d_ref, md, par, it):
        masked = jnp.where(it, INF, md)
        m = jnp.min(masked)
        v = jnp.min(jnp.where(masked == m, iota, BIG))
        row = d_ref[v]
        it_n = it | (iota == v)
        upd = (row < md) & jnp.logical_not(it_n)
        par_n = jnp.where(upd, v, par)
        md_n = jnp.where(upd, row, md)
        return md_n, par_n, it_n

    def body(i, st):
        md1, par1, it1, md2, par2, it2 = st
        md1, par1, it1 = step(d1_ref, md1, par1, it1)
        md2, par2, it2 = step(d2_ref, md2, par2, it2)
        return (md1, par1, it1, md2, par2, it2)

    zero_i = jnp.zeros((S, L), jnp.int32)
    init = (d1_ref[0], zero_i, root, d2_ref[0], zero_i, root)
    md1, par1, _, md2, par2, _ = lax.fori_loop(0, n - 1, body, init)

    nonroot = jnp.logical_not(root)
    matched = jnp.sum(jnp.where((par1 == par2) & nonroot, 1.0, 0.0))

    def gbody(ub, accs):
        a12, a21 = accs
        base = ub * 8
        blk2 = d2_ref[pl.ds(base, 8)]
        blk1 = d1_ref[pl.ds(base, 8)]
        for du in range(8):
            u = base + du
            a12 = a12 + jnp.where(par1 == u, blk2[du], 0.0)
            a21 = a21 + jnp.where(par2 == u, blk1[du], 0.0)
        return (a12, a21)

    zero_f = jnp.zeros((S, L), jnp.float32)
    sig12, sig21 = lax.fori_loop(0, n // 8, gbody, (zero_f, zero_f))

    d12 = jnp.sqrt(jnp.sum(jnp.where(nonroot, (md1 - sig12) ** 2, 0.0)))
    d21 = jnp.sqrt(jnp.sum(jnp.where(nonroot, (md2 - sig21) ** 2, 0.0)))

    dist_ref[0, 0] = d12 + d21
    matched_ref[0, 0] = matched
    d12_ref[0, 0] = d12
    d21_ref[0, 0] = d21


def kernel(distances1, distances2):
    n = distances1.shape[0]
    d1r = distances1.reshape(n, n // 128, 128)
    d2r = distances2.reshape(n, n // 128, 128)
    out_shape = [jax.ShapeDtypeStruct((1, 1), jnp.float32)] * 4
    dist, matched, d12, d21 = pl.pallas_call(
        _tsd_kernel, out_shape=out_shape)(d1r, d2r)
    return (dist[0, 0], matched[0, 0], d12[0, 0], d21[0, 0])


# speedup vs baseline: 45.9421x; 45.9421x over previous
import jax
import jax.numpy as jnp
from jax import lax
from jax.experimental import pallas as pl


def _tsd_kernel(d1_ref, d2_ref, dist_ref, matched_ref, d12_ref, d21_ref):
    n, S, L = d1_ref.shape
    iota = (lax.broadcasted_iota(jnp.int32, (S, L), 0) * L
            + lax.broadcasted_iota(jnp.int32, (S, L), 1))
    INF = jnp.float32(jnp.inf)
    BIG = jnp.int32(1 << 30)
    root = iota == 0

    def step(d_ref, md, par, it):
        masked = jnp.where(it != 0, INF, md)
        m = jnp.min(masked)
        v = jnp.min(jnp.where(masked == m, iota, BIG))
        row = d_ref[v]
        it_n = it | (iota == v).astype(jnp.int32)
        upd = (row < md) & (it_n == 0)
        par_n = jnp.where(upd, v, par)
        md_n = jnp.where(upd, row, md)
        return md_n, par_n, it_n

    def body(i, st):
        md1, par1, it1, md2, par2, it2 = st
        md1, par1, it1 = step(d1_ref, md1, par1, it1)
        md2, par2, it2 = step(d2_ref, md2, par2, it2)
        return (md1, par1, it1, md2, par2, it2)

    zero_i = jnp.zeros((S, L), jnp.int32)
    root_i = root.astype(jnp.int32)
    init = (d1_ref[0], zero_i, root_i, d2_ref[0], zero_i, root_i)
    md1, par1, _, md2, par2, _ = lax.fori_loop(0, n - 1, body, init)

    nonroot = jnp.logical_not(root)
    matched = jnp.sum(jnp.where((par1 == par2) & nonroot, 1.0, 0.0))

    def gbody(ub, accs):
        a12, a21 = accs
        base = ub * 8
        blk2 = d2_ref[pl.ds(base, 8)]
        blk1 = d1_ref[pl.ds(base, 8)]
        for du in range(8):
            u = base + du
            a12 = a12 + jnp.where(par1 == u, blk2[du], 0.0)
            a21 = a21 + jnp.where(par2 == u, blk1[du], 0.0)
        return (a12, a21)

    zero_f = jnp.zeros((S, L), jnp.float32)
    sig12, sig21 = lax.fori_loop(0, n // 8, gbody, (zero_f, zero_f))

    d12 = jnp.sqrt(jnp.sum(jnp.where(nonroot, (md1 - sig12) ** 2, 0.0)))
    d21 = jnp.sqrt(jnp.sum(jnp.where(nonroot, (md2 - sig21) ** 2, 0.0)))

    dist_ref[...] = (d12 + d21).reshape(1, 1)
    matched_ref[...] = matched.reshape(1, 1)
    d12_ref[...] = d12.reshape(1, 1)
    d21_ref[...] = d21.reshape(1, 1)


def kernel(distances1, distances2):
    n = distances1.shape[0]
    d1r = distances1.reshape(n, n // 128, 128)
    d2r = distances2.reshape(n, n // 128, 128)
    out_shape = [jax.ShapeDtypeStruct((1, 1), jnp.float32)] * 4
    dist, matched, d12, d21 = pl.pallas_call(
        _tsd_kernel, out_shape=out_shape)(d1r, d2r)
    return (dist[0, 0], matched[0, 0], d12[0, 0], d21[0, 0])


# speedup vs baseline: 172.8594x; 3.7625x over previous
import jax
import jax.numpy as jnp
from jax import lax
from jax.experimental import pallas as pl


def _tsd_kernel(d1_ref, d2_ref, dist_ref, matched_ref, d12_ref, d21_ref):
    n, S, L = d1_ref.shape
    iota = (lax.broadcasted_iota(jnp.int32, (S, L), 0) * L
            + lax.broadcasted_iota(jnp.int32, (S, L), 1))
    INF = jnp.float32(jnp.inf)
    BIG = jnp.int32(1 << 30)
    root = iota == 0

    sub_iota = lax.broadcasted_iota(jnp.int32, (S, 1), 0)

    def step(d_ref, md, par, it):
        masked = jnp.where(it != 0, INF, md)
        lane_idx = jnp.argmin(masked, axis=1).astype(jnp.int32).reshape(S, 1)
        lane_min = jnp.min(masked, axis=1).reshape(S, 1)
        lin = sub_iota * L + lane_idx
        lm = lane_min
        for sh in (4, 2, 1):
            lm_r = jnp.roll(lm, sh, axis=0)
            lin_r = jnp.roll(lin, sh, axis=0)
            take = (lm_r < lm) | ((lm_r == lm) & (lin_r < lin))
            lm = jnp.where(take, lm_r, lm)
            lin = jnp.where(take, lin_r, lin)
        v = lin[0, 0]
        row = d_ref[v]
        it_n = it | (iota == v).astype(jnp.int32)
        upd = (row < md) & (it_n == 0)
        par_n = jnp.where(upd, v, par)
        md_n = jnp.where(upd, row, md)
        return md_n, par_n, it_n

    def body(i, st):
        md1, par1, it1, md2, par2, it2 = st
        md1, par1, it1 = step(d1_ref, md1, par1, it1)
        md2, par2, it2 = step(d2_ref, md2, par2, it2)
        return (md1, par1, it1, md2, par2, it2)

    zero_i = jnp.zeros((S, L), jnp.int32)
    root_i = root.astype(jnp.int32)
    init = (d1_ref[0], zero_i, root_i, d2_ref[0], zero_i, root_i)
    md1, par1, _, md2, par2, _ = lax.fori_loop(0, n - 1, body, init)

    nonroot = jnp.logical_not(root)
    matched = jnp.sum(jnp.where((par1 == par2) & nonroot, 1.0, 0.0))

    def gbody(ub, accs):
        a12, a21 = accs
        base = ub * 8
        blk2 = d2_ref[pl.ds(base, 8)]
        blk1 = d1_ref[pl.ds(base, 8)]
        for du in range(8):
            u = base + du
            a12 = a12 + jnp.where(par1 == u, blk2[du], 0.0)
            a21 = a21 + jnp.where(par2 == u, blk1[du], 0.0)
        return (a12, a21)

    zero_f = jnp.zeros((S, L), jnp.float32)
    sig12, sig21 = lax.fori_loop(0, n // 8, gbody, (zero_f, zero_f))

    d12 = jnp.sqrt(jnp.sum(jnp.where(nonroot, (md1 - sig12) ** 2, 0.0)))
    d21 = jnp.sqrt(jnp.sum(jnp.where(nonroot, (md2 - sig21) ** 2, 0.0)))

    dist_ref[...] = (d12 + d21).reshape(1, 1)
    matched_ref[...] = matched.reshape(1, 1)
    d12_ref[...] = d12.reshape(1, 1)
    d21_ref[...] = d21.reshape(1, 1)


def kernel(distances1, distances2):
    n = distances1.shape[0]
    d1r = distances1.reshape(n, n // 128, 128)
    d2r = distances2.reshape(n, n // 128, 128)
    out_shape = [jax.ShapeDtypeStruct((1, 1), jnp.float32)] * 4
    dist, matched, d12, d21 = pl.pallas_call(
        _tsd_kernel, out_shape=out_shape)(d1r, d2r)
    return (dist[0, 0], matched[0, 0], d12[0, 0], d21[0, 0])


# speedup vs baseline: 174.2370x; 1.0080x over previous
import jax
import jax.numpy as jnp
from jax import lax
from jax.experimental import pallas as pl


def _tsd_kernel(d1_ref, d2_ref, dist_ref, matched_ref, d12_ref, d21_ref):
    n, S, L = d1_ref.shape
    iota = (lax.broadcasted_iota(jnp.int32, (S, L), 0) * L
            + lax.broadcasted_iota(jnp.int32, (S, L), 1))
    INF = jnp.float32(jnp.inf)
    BIG = jnp.int32(1 << 30)
    root = iota == 0

    sub_iota = lax.broadcasted_iota(jnp.int32, (S, 1), 0)

    def step(d_ref, masked, md, par):
        notin = masked < INF
        lane_idx = jnp.argmin(masked, axis=1).astype(jnp.int32).reshape(S, 1)
        lane_min = jnp.min(masked, axis=1).reshape(S, 1)
        lin = sub_iota * L + lane_idx
        lm = lane_min
        for sh in (4, 2, 1):
            lm_r = jnp.roll(lm, sh, axis=0)
            lin_r = jnp.roll(lin, sh, axis=0)
            take = (lm_r < lm) | ((lm_r == lm) & (lin_r < lin))
            lm = jnp.where(take, lm_r, lm)
            lin = jnp.where(take, lin_r, lin)
        v = lin[0, 0]
        row = d_ref[v]
        is_v = iota == v
        better = (row < masked) & notin
        masked_n = jnp.where(is_v, INF, jnp.where(better, row, masked))
        upd = better & jnp.logical_not(is_v)
        par_n = jnp.where(upd, v, par)
        md_n = jnp.where(upd, row, md)
        return masked_n, md_n, par_n

    def body(i, st):
        ma1, md1, par1, ma2, md2, par2 = st
        ma1, md1, par1 = step(d1_ref, ma1, md1, par1)
        ma2, md2, par2 = step(d2_ref, ma2, md2, par2)
        return (ma1, md1, par1, ma2, md2, par2)

    zero_i = jnp.zeros((S, L), jnp.int32)
    row0_1 = d1_ref[0]
    row0_2 = d2_ref[0]
    init = (jnp.where(root, INF, row0_1), row0_1, zero_i,
            jnp.where(root, INF, row0_2), row0_2, zero_i)
    _, md1, par1, _, md2, par2 = lax.fori_loop(0, n - 1, body, init)

    nonroot = jnp.logical_not(root)
    matched = jnp.sum(jnp.where((par1 == par2) & nonroot, 1.0, 0.0))

    def gbody(ub, accs):
        a12, a21 = accs
        base = ub * 8
        blk2 = d2_ref[pl.ds(base, 8)]
        blk1 = d1_ref[pl.ds(base, 8)]
        for du in range(8):
            u = base + du
            a12 = a12 + jnp.where(par1 == u, blk2[du], 0.0)
            a21 = a21 + jnp.where(par2 == u, blk1[du], 0.0)
        return (a12, a21)

    zero_f = jnp.zeros((S, L), jnp.float32)
    sig12, sig21 = lax.fori_loop(0, n // 8, gbody, (zero_f, zero_f))

    d12 = jnp.sqrt(jnp.sum(jnp.where(nonroot, (md1 - sig12) ** 2, 0.0)))
    d21 = jnp.sqrt(jnp.sum(jnp.where(nonroot, (md2 - sig21) ** 2, 0.0)))

    dist_ref[...] = (d12 + d21).reshape(1, 1)
    matched_ref[...] = matched.reshape(1, 1)
    d12_ref[...] = d12.reshape(1, 1)
    d21_ref[...] = d21.reshape(1, 1)


def kernel(distances1, distances2):
    n = distances1.shape[0]
    d1r = distances1.reshape(n, n // 128, 128)
    d2r = distances2.reshape(n, n // 128, 128)
    out_shape = [jax.ShapeDtypeStruct((1, 1), jnp.float32)] * 4
    dist, matched, d12, d21 = pl.pallas_call(
        _tsd_kernel, out_shape=out_shape)(d1r, d2r)
    return (dist[0, 0], matched[0, 0], d12[0, 0], d21[0, 0])
